# baseline (device time: 234667 ns/iter reference)
import jax
import jax.numpy as jnp
from jax import lax
from jax.experimental import pallas as pl
from jax.experimental.pallas import tpu as pltpu

N_DEV = 8
B = 2
SQ = 512
DMODEL = 768
HQ = 8
DH = 64
DF = HQ * DH
SKV_SHARD = 512
SCALE = 0.125
NEG = -1e9


def kernel(x, Wq, K_ext, V_ext, Wo):
    def body(x_ref, wq_ref, k_ref, v_ref, wo_ref, out_ref,
             comm_o, comm_m, comm_l, send_sems, recv_sems):
        my = lax.axis_index("i")
        left = lax.rem(my - 1 + N_DEV, N_DEV)
        right = lax.rem(my + 1, N_DEV)

        barrier_sem = pltpu.get_barrier_semaphore()
        for nbr in (left, right):
            pl.semaphore_signal(
                barrier_sem, inc=1,
                device_id=(nbr,), device_id_type=pl.DeviceIdType.MESH,
            )
        pl.semaphore_wait(barrier_sem, 2)

        qi = lax.broadcasted_iota(jnp.int32, (SQ, SKV_SHARD), 0)
        kj = lax.broadcasted_iota(jnp.int32, (SQ, SKV_SHARD), 1) + my * SKV_SHARD
        mask = (jnp.abs(qi - kj) <= 128) | (kj < 32) | (qi < 32)

        M = [[None] * HQ for _ in range(B)]
        L = [[None] * HQ for _ in range(B)]
        O = [[None] * HQ for _ in range(B)]
        for b in range(B):
            q_b = jnp.dot(x_ref[b], wq_ref[:, :],
                          preferred_element_type=jnp.float32)
            for h in range(HQ):
                q_bh = q_b[:, h * DH:(h + 1) * DH]
                k_bh = k_ref[b, :, h, :]
                v_bh = v_ref[b, :, h, :]
                s = lax.dot_general(
                    q_bh, k_bh, (((1,), (1,)), ((), ())),
                    preferred_element_type=jnp.float32) * SCALE
                s = jnp.where(mask, s, NEG)
                m = jnp.max(s, axis=1, keepdims=True)
                w = jnp.exp(s - m)
                l = jnp.sum(w, axis=1, keepdims=True)
                o = jnp.dot(w, v_bh, preferred_element_type=jnp.float32)
                M[b][h] = m
                L[b][h] = l
                O[b][h] = o

        for b in range(B):
            comm_o[0, b, :, :] = jnp.concatenate(
                [O[b][h] for h in range(HQ)], axis=1)
            for h in range(HQ):
                comm_m[0, b, h, :] = M[b][h][:, 0]
                comm_l[0, b, h, :] = L[b][h][:, 0]

        for hop in range(N_DEV - 1):
            ss = hop % 2
            rs = (hop + 1) % 2
            copies = []
            for t, buf in enumerate((comm_o, comm_m, comm_l)):
                c = pltpu.make_async_remote_copy(
                    src_ref=buf.at[ss],
                    dst_ref=buf.at[rs],
                    send_sem=send_sems.at[t, ss],
                    recv_sem=recv_sems.at[t, rs],
                    device_id=(right,),
                    device_id_type=pl.DeviceIdType.MESH,
                )
                c.start()
                copies.append(c)
            for c in copies:
                c.wait()

            for b in range(B):
                o_all = comm_o[rs, b, :, :]
                for h in range(HQ):
                    mj = comm_m[rs, b, h, :][:, None]
                    lj = comm_l[rs, b, h, :][:, None]
                    oj = o_all[:, h * DH:(h + 1) * DH]
                    m_new = jnp.maximum(M[b][h], mj)
                    alpha = jnp.exp(M[b][h] - m_new)
                    beta = jnp.exp(mj - m_new)
                    L[b][h] = L[b][h] * alpha + lj * beta
                    O[b][h] = O[b][h] * alpha + oj * beta
                    M[b][h] = m_new

        for b in range(B):
            ctx = jnp.concatenate(
                [O[b][h] / L[b][h] for h in range(HQ)], axis=1)
            out_ref[b, :, :] = jnp.dot(
                ctx, wo_ref[:, :], preferred_element_type=jnp.float32)

    return pl.pallas_call(
        body,
        out_shape=jax.ShapeDtypeStruct((B, SQ, DMODEL), jnp.float32),
        in_specs=[pl.BlockSpec(memory_space=pltpu.VMEM)] * 5,
        out_specs=pl.BlockSpec(memory_space=pltpu.VMEM),
        scratch_shapes=[
            pltpu.VMEM((2, B, SQ, DF), jnp.float32),
            pltpu.VMEM((2, B, HQ, SQ), jnp.float32),
            pltpu.VMEM((2, B, HQ, SQ), jnp.float32),
            pltpu.SemaphoreType.DMA((3, 2)),
            pltpu.SemaphoreType.DMA((3, 2)),
        ],
        compiler_params=pltpu.CompilerParams(collective_id=0),
    )(x, Wq, K_ext, V_ext, Wo)


# device time: 90326 ns/iter; 2.5980x vs baseline; 2.5980x over previous
import jax
import jax.numpy as jnp
from jax import lax
from jax.experimental import pallas as pl
from jax.experimental.pallas import tpu as pltpu

N_DEV = 8
B = 2
SQ = 512
DMODEL = 768
HQ = 8
DH = 64
DF = HQ * DH
PACK = DF + 128
SKV_SHARD = 512
SCALE = 0.125
NEG = -1e9
ROUNDS = (1, 3, 4)


def kernel(x, Wq, K_ext, V_ext, Wo):
    K2 = K_ext.reshape(B, SKV_SHARD, DF)
    V2 = V_ext.reshape(B, SKV_SHARD, DF)

    def body(x_ref, wq_ref, k_ref, v_ref, wo_ref, out_ref,
             comm_send, comm_recv, send_sems, recv_sems):
        my = lax.axis_index("i")

        barrier_sem = pltpu.get_barrier_semaphore()
        for xr in ROUNDS:
            pl.semaphore_signal(
                barrier_sem, inc=1,
                device_id=(my ^ xr,), device_id_type=pl.DeviceIdType.MESH,
            )
        pl.semaphore_wait(barrier_sem, 3)

        qi = lax.broadcasted_iota(jnp.int32, (SQ, SKV_SHARD), 0)
        kj = lax.broadcasted_iota(jnp.int32, (SQ, SKV_SHARD), 1) + my * SKV_SHARD
        mask = (jnp.abs(qi - kj) <= 128) | (kj < 32) | (qi < 32)

        M = [[None] * HQ for _ in range(B)]
        L = [[None] * HQ for _ in range(B)]
        O = [[None] * HQ for _ in range(B)]

        def pack(b, h):
            comm_send[b, :, h * DH:(h + 1) * DH] = O[b][h].astype(jnp.bfloat16)
            comm_send[b, :, DF + h:DF + h + 1] = M[b][h].astype(jnp.bfloat16)
            comm_send[b, :, DF + 8 + h:DF + 9 + h] = L[b][h].astype(jnp.bfloat16)

        for b in range(B):
            q_b = jnp.dot(x_ref[b], wq_ref[:, :],
                          preferred_element_type=jnp.float32)
            for h in range(HQ):
                q_bh = q_b[:, h * DH:(h + 1) * DH]
                k_bh = k_ref[b, :, h * DH:(h + 1) * DH]
                v_bh = v_ref[b, :, h * DH:(h + 1) * DH]
                s = lax.dot_general(
                    q_bh, k_bh, (((1,), (1,)), ((), ())),
                    preferred_element_type=jnp.float32) * SCALE
                s = jnp.where(mask, s, NEG)
                m = jnp.max(s, axis=1, keepdims=True)
                w = jnp.exp(s - m)
                l = jnp.sum(w, axis=1, keepdims=True)
                o = jnp.dot(w, v_bh, preferred_element_type=jnp.float32)
                M[b][h] = m
                L[b][h] = l
                O[b][h] = o
                pack(b, h)

        for r, xr in enumerate(ROUNDS):
            partner = my ^ xr
            rdma = pltpu.make_async_remote_copy(
                src_ref=comm_send,
                dst_ref=comm_recv.at[r],
                send_sem=send_sems.at[r],
                recv_sem=recv_sems.at[r],
                device_id=(partner,),
                device_id_type=pl.DeviceIdType.MESH,
            )
            rdma.start()
            rdma.wait()

            for b in range(B):
                for h in range(HQ):
                    oj = comm_recv[r, b, :, h * DH:(h + 1) * DH].astype(jnp.float32)
                    mj = comm_recv[r, b, :, DF + h:DF + h + 1].astype(jnp.float32)
                    lj = comm_recv[r, b, :, DF + 8 + h:DF + 9 + h].astype(jnp.float32)
                    m_new = jnp.maximum(M[b][h], mj)
                    alpha = jnp.exp(M[b][h] - m_new)
                    beta = jnp.exp(mj - m_new)
                    L[b][h] = L[b][h] * alpha + lj * beta
                    O[b][h] = O[b][h] * alpha + oj * beta
                    M[b][h] = m_new
                    if r < len(ROUNDS) - 1:
                        pack(b, h)

        for b in range(B):
            ctx = jnp.concatenate(
                [O[b][h] / L[b][h] for h in range(HQ)], axis=1)
            out_ref[b, :, :] = jnp.dot(
                ctx, wo_ref[:, :], preferred_element_type=jnp.float32)

    return pl.pallas_call(
        body,
        out_shape=jax.ShapeDtypeStruct((B, SQ, DMODEL), jnp.float32),
        in_specs=[pl.BlockSpec(memory_space=pltpu.VMEM)] * 5,
        out_specs=pl.BlockSpec(memory_space=pltpu.VMEM),
        scratch_shapes=[
            pltpu.VMEM((B, SQ, PACK), jnp.bfloat16),
            pltpu.VMEM((3, B, SQ, PACK), jnp.bfloat16),
            pltpu.SemaphoreType.DMA((3,)),
            pltpu.SemaphoreType.DMA((3,)),
        ],
        compiler_params=pltpu.CompilerParams(collective_id=0),
    )(x, Wq, K2, V2, Wo)


# device time: 70374 ns/iter; 3.3346x vs baseline; 1.2835x over previous
import jax
import jax.numpy as jnp
from jax import lax
from jax.experimental import pallas as pl
from jax.experimental.pallas import tpu as pltpu

N_DEV = 8
B = 2
SQ = 512
DMODEL = 768
HQ = 8
DH = 64
DF = HQ * DH
PACK = DF + 128
SKV_SHARD = 512
SCALE = 0.125
NEG = -1e9
ROUNDS = (1, 3, 4)


def kernel(x, Wq, K_ext, V_ext, Wo):
    K2 = K_ext.reshape(B, SKV_SHARD, DF)
    V2 = V_ext.reshape(B, SKV_SHARD, DF)

    def body(x_ref, wq_ref, k_ref, v_ref, wo_ref, out_ref,
             comm_send, comm_recv, send_sems, recv_sems):
        my = lax.axis_index("i")

        barrier_sem = pltpu.get_barrier_semaphore()
        for xr in ROUNDS:
            pl.semaphore_signal(
                barrier_sem, inc=1,
                device_id=(my ^ xr,), device_id_type=pl.DeviceIdType.MESH,
            )
        pl.semaphore_wait(barrier_sem, 3)

        qi = lax.broadcasted_iota(jnp.int32, (SQ, SKV_SHARD), 0)
        kj = lax.broadcasted_iota(jnp.int32, (SQ, SKV_SHARD), 1) + my * SKV_SHARD
        mask = (jnp.abs(qi - kj) <= 128) | (kj < 32) | (qi < 32)

        L = [[None] * HQ for _ in range(B)]
        O = [[None] * HQ for _ in range(B)]

        def pack(b, h):
            comm_send[b, :, h * DH:(h + 1) * DH] = O[b][h].astype(jnp.bfloat16)
            comm_send[b, :, DF + h:DF + h + 1] = L[b][h].astype(jnp.bfloat16)

        for b in range(B):
            q_b = jnp.dot(x_ref[b], wq_ref[:, :],
                          preferred_element_type=jnp.float32)
            for h in range(HQ):
                q_bh = q_b[:, h * DH:(h + 1) * DH]
                k_bh = k_ref[b, :, h * DH:(h + 1) * DH]
                v_bh = v_ref[b, :, h * DH:(h + 1) * DH]
                s = lax.dot_general(
                    q_bh, k_bh, (((1,), (1,)), ((), ())),
                    preferred_element_type=jnp.float32) * SCALE
                w = jnp.exp(jnp.where(mask, s, NEG))
                l = jnp.sum(w, axis=1, keepdims=True)
                o = jnp.dot(w, v_bh, preferred_element_type=jnp.float32)
                L[b][h] = l
                O[b][h] = o
                pack(b, h)

        for r, xr in enumerate(ROUNDS):
            partner = my ^ xr
            rdma = pltpu.make_async_remote_copy(
                src_ref=comm_send,
                dst_ref=comm_recv.at[r],
                send_sem=send_sems.at[r],
                recv_sem=recv_sems.at[r],
                device_id=(partner,),
                device_id_type=pl.DeviceIdType.MESH,
            )
            rdma.start()
            rdma.wait()

            for b in range(B):
                for h in range(HQ):
                    oj = comm_recv[r, b, :, h * DH:(h + 1) * DH].astype(jnp.float32)
                    lj = comm_recv[r, b, :, DF + h:DF + h + 1].astype(jnp.float32)
                    L[b][h] = L[b][h] + lj
                    O[b][h] = O[b][h] + oj
                    if r < len(ROUNDS) - 1:
                        pack(b, h)

        for b in range(B):
            ctx = jnp.concatenate(
                [O[b][h] * (1.0 / L[b][h]) for h in range(HQ)], axis=1)
            out_ref[b, :, :] = jnp.dot(
                ctx, wo_ref[:, :], preferred_element_type=jnp.float32)

    return pl.pallas_call(
        body,
        out_shape=jax.ShapeDtypeStruct((B, SQ, DMODEL), jnp.float32),
        in_specs=[pl.BlockSpec(memory_space=pltpu.VMEM)] * 5,
        out_specs=pl.BlockSpec(memory_space=pltpu.VMEM),
        scratch_shapes=[
            pltpu.VMEM((B, SQ, PACK), jnp.bfloat16),
            pltpu.VMEM((3, B, SQ, PACK), jnp.bfloat16),
            pltpu.SemaphoreType.DMA((3,)),
            pltpu.SemaphoreType.DMA((3,)),
        ],
        compiler_params=pltpu.CompilerParams(collective_id=0),
    )(x, Wq, K2, V2, Wo)


# device time: 49418 ns/iter; 4.7486x vs baseline; 1.4241x over previous
import jax
import jax.numpy as jnp
from jax import lax
from jax.experimental import pallas as pl
from jax.experimental.pallas import tpu as pltpu

N_DEV = 8
B = 2
SQ = 512
DMODEL = 768
HQ = 8
DH = 64
DF = HQ * DH
PACK = DF + 128
SKV_SHARD = 512
SCALE = 0.125
NEG = -1e9
ROUNDS = (1, 3, 4)
NC = 4
CH = SQ // NC


def kernel(x, Wq, K_ext, V_ext, Wo):
    K2 = K_ext.reshape(B, SKV_SHARD, DF)
    V2 = V_ext.reshape(B, SKV_SHARD, DF)

    def body(x_ref, wq_ref, k_ref, v_ref, wo_ref, out_ref,
             comm_send, comm_recv, send_sems, recv_sems):
        my = lax.axis_index("i")

        barrier_sem = pltpu.get_barrier_semaphore()
        for xr in ROUNDS:
            pl.semaphore_signal(
                barrier_sem, inc=1,
                device_id=(my ^ xr,), device_id_type=pl.DeviceIdType.MESH,
            )
        pl.semaphore_wait(barrier_sem, 3)

        kj = lax.broadcasted_iota(jnp.int32, (CH, SKV_SHARD), 1) + my * SKV_SHARD

        L = [[[None] * HQ for _ in range(B)] for _ in range(NC)]
        O = [[[None] * HQ for _ in range(B)] for _ in range(NC)]
        rdmas = [[None] * NC for _ in range(len(ROUNDS))]

        def pack(c, b, h):
            comm_send[c, b, :, h * DH:(h + 1) * DH] = O[c][b][h].astype(jnp.bfloat16)
            comm_send[c, b, :, DF + h:DF + h + 1] = L[c][b][h].astype(jnp.bfloat16)

        def start_round(r, c):
            rdma = pltpu.make_async_remote_copy(
                src_ref=comm_send.at[c],
                dst_ref=comm_recv.at[r, c],
                send_sem=send_sems.at[r, c],
                recv_sem=recv_sems.at[r, c],
                device_id=(my ^ ROUNDS[r],),
                device_id_type=pl.DeviceIdType.MESH,
            )
            rdma.start()
            rdmas[r][c] = rdma

        for c in range(NC):
            qi = lax.broadcasted_iota(jnp.int32, (CH, SKV_SHARD), 0) + c * CH
            mask = (jnp.abs(qi - kj) <= 128) | (kj < 32) | (qi < 32)
            for b in range(B):
                q_bc = jnp.dot(x_ref[b, c * CH:(c + 1) * CH, :], wq_ref[:, :],
                               preferred_element_type=jnp.float32)
                for h in range(HQ):
                    q_bh = q_bc[:, h * DH:(h + 1) * DH]
                    k_bh = k_ref[b, :, h * DH:(h + 1) * DH]
                    v_bh = v_ref[b, :, h * DH:(h + 1) * DH]
                    s = lax.dot_general(
                        q_bh, k_bh, (((1,), (1,)), ((), ())),
                        preferred_element_type=jnp.float32) * SCALE
                    w = jnp.exp(jnp.where(mask, s, NEG))
                    l = jnp.sum(w, axis=1, keepdims=True)
                    o = jnp.dot(w, v_bh, preferred_element_type=jnp.float32)
                    L[c][b][h] = l
                    O[c][b][h] = o
                    pack(c, b, h)
            start_round(0, c)

        for r in range(1, len(ROUNDS)):
            for c in range(NC):
                rdmas[r - 1][c].wait()
                for b in range(B):
                    for h in range(HQ):
                        oj = comm_recv[r - 1, c, b, :, h * DH:(h + 1) * DH]
                        lj = comm_recv[r - 1, c, b, :, DF + h:DF + h + 1]
                        L[c][b][h] = L[c][b][h] + lj.astype(jnp.float32)
                        O[c][b][h] = O[c][b][h] + oj.astype(jnp.float32)
                        pack(c, b, h)
                start_round(r, c)

        last = len(ROUNDS) - 1
        for c in range(NC):
            rdmas[last][c].wait()
            for b in range(B):
                for h in range(HQ):
                    oj = comm_recv[last, c, b, :, h * DH:(h + 1) * DH]
                    lj = comm_recv[last, c, b, :, DF + h:DF + h + 1]
                    L[c][b][h] = L[c][b][h] + lj.astype(jnp.float32)
                    O[c][b][h] = O[c][b][h] + oj.astype(jnp.float32)
                ctx = jnp.concatenate(
                    [O[c][b][h] * (1.0 / L[c][b][h]) for h in range(HQ)],
                    axis=1)
                out_ref[b, c * CH:(c + 1) * CH, :] = jnp.dot(
                    ctx, wo_ref[:, :], preferred_element_type=jnp.float32)

    return pl.pallas_call(
        body,
        out_shape=jax.ShapeDtypeStruct((B, SQ, DMODEL), jnp.float32),
        in_specs=[pl.BlockSpec(memory_space=pltpu.VMEM)] * 5,
        out_specs=pl.BlockSpec(memory_space=pltpu.VMEM),
        scratch_shapes=[
            pltpu.VMEM((NC, B, CH, PACK), jnp.bfloat16),
            pltpu.VMEM((3, NC, B, CH, PACK), jnp.bfloat16),
            pltpu.SemaphoreType.DMA((3, NC)),
            pltpu.SemaphoreType.DMA((3, NC)),
        ],
        compiler_params=pltpu.CompilerParams(collective_id=0),
    )(x, Wq, K2, V2, Wo)
